# Initial kernel scaffold; baseline (speedup 1.0000x reference)
import jax
import jax.numpy as jnp
from jax.experimental import pallas as pl


def kernel(x, W_in, bn1_gamma, bn1_beta, h_s1_Wl, h_s1_bl, h_s1_Wr, h_s2_Wl, h_s2_bl, h_s2_Wr, bn2_gamma, bn2_beta, o_s1_Wl, o_s1_bl, o_s1_Wr, o_s2_Wl, o_s2_bl, o_s2_Wr, o_sc_W, o_sc_b):
    raise NotImplementedError("write your pallas kernel here")


# speedup vs baseline: 43.3415x; 43.3415x over previous
import functools

import jax
import jax.numpy as jnp
from jax import lax
from jax.experimental import pallas as pl
from jax.experimental.pallas import tpu as pltpu
from jax.experimental.pallas import tpu_sc as plsc

N = 4096
D = 256
H = 256
NTOT = N * N
KIDX = 15938354
BRANK = NTOT - KIDX
BRANK_F = float(BRANK)

NB = 32768
NBA = 32896
NROWS = NBA // 128
SHIFT1 = 15

NC = 2
NS = 16
NW = NC * NS
CHUNK = NTOT // NW
SLICE = 32768
NSLICE = CHUNK // SLICE
UNROLL = 8

RB = 512
NBLK = N // RB


def _make_hist(shift):
    mesh = plsc.VectorSubcoreMesh(
        core_axis_name="c", subcore_axis_name="s", num_cores=NC, num_subcores=NS
    )

    @functools.partial(
        pl.kernel,
        out_type=jax.ShapeDtypeStruct((NW, NBA), jnp.int32),
        mesh=mesh,
        compiler_params=pltpu.CompilerParams(needs_layout_passes=False),
        scratch_types=[
            pltpu.VMEM((NBA,), jnp.int32),
            pltpu.VMEM((SLICE,), jnp.int32),
            pltpu.VMEM((16,), jnp.int32),
        ],
    )
    def hist_k(sim_hbm, base_hbm, out_hbm, hist_v, buf_v, base_v):
        wid = lax.axis_index("s") * NC + lax.axis_index("c")

        def zbody(i, carry):
            hist_v[pl.ds(i * 16, 16)] = jnp.zeros((16,), jnp.int32)
            return carry

        lax.fori_loop(0, NBA // 16, zbody, 0)

        pltpu.sync_copy(base_hbm, base_v)
        base = base_v[...]
        ones = jnp.ones((16,), jnp.int32)
        chunk0 = wid * CHUNK

        def sbody(s, carry):
            pltpu.sync_copy(sim_hbm.at[pl.ds(chunk0 + s * SLICE, SLICE)], buf_v)

            def ibody(i, c2):
                for u in range(UNROLL):
                    bits = buf_v[pl.ds((i * UNROLL + u) * 16, 16)]
                    d = bits - base
                    if shift:
                        d = d >> shift
                    b = jnp.clip(d + 1, 0, NB + 1)
                    plsc.addupdate_scatter(hist_v, [b], ones)
                return c2

            lax.fori_loop(0, SLICE // (16 * UNROLL), ibody, 0)
            return carry

        lax.fori_loop(0, NSLICE, sbody, 0)
        pltpu.sync_copy(hist_v, out_hbm.at[wid])

    return hist_k


@functools.lru_cache(maxsize=None)
def _hist_fns():
    return _make_hist(SHIFT1), _make_hist(0)


def _make_pick(shift):
    def pick_body(hist_ref, base_ref, nb_ref, thr_ref):
        tot = hist_ref[0]
        for r in range(1, NW):
            tot = tot + hist_ref[r]
        tot = tot.astype(jnp.float32)

        li = lax.broadcasted_iota(jnp.int32, (128, 128), 0)
        lj = lax.broadcasted_iota(jnp.int32, (128, 128), 1)
        ltri = (li >= lj).astype(jnp.float32)
        sfx_lane = jnp.dot(tot, ltri, preferred_element_type=jnp.float32)

        ri = lax.broadcasted_iota(jnp.int32, (NROWS, NROWS), 0)
        rj = lax.broadcasted_iota(jnp.int32, (NROWS, NROWS), 1)
        sutri = (rj > ri).astype(jnp.float32)
        rows_after = jnp.sum(
            jnp.dot(sutri, tot, preferred_element_type=jnp.float32),
            axis=1, keepdims=True,
        )
        sfx = sfx_lane + rows_after

        jr = lax.broadcasted_iota(jnp.int32, (NROWS, 128), 0)
        jc = lax.broadcasted_iota(jnp.int32, (NROWS, 128), 1)
        jf = (jr * 128 + jc).astype(jnp.float32)
        best = jnp.max(jnp.where(sfx >= BRANK_F, jf, -1.0))
        jstar = best.astype(jnp.int32)

        base0 = base_ref[0]
        newbase = base0 + ((jstar - 1) << shift) if shift else base0 + (jstar - 1)
        nb_ref[...] = jnp.full((16,), newbase, jnp.int32)
        thr_ref[...] = jnp.full((1, 128), newbase, jnp.int32)

    def pick(hists, base16):
        return pl.pallas_call(
            pick_body,
            grid=(),
            in_specs=[
                pl.BlockSpec((NW, NROWS, 128), lambda: (0, 0, 0)),
                pl.BlockSpec((16,), lambda: (0,)),
            ],
            out_specs=[
                pl.BlockSpec((16,), lambda: (0,)),
                pl.BlockSpec((1, 128), lambda: (0, 0)),
            ],
            out_shape=[
                jax.ShapeDtypeStruct((16,), jnp.int32),
                jax.ShapeDtypeStruct((1, 128), jnp.int32),
            ],
        )(hists.reshape(NW, NROWS, 128), base16)

    return pick


_pick_p1 = _make_pick(SHIFT1)
_pick_p2 = _make_pick(0)


def _dot_t(a, b):
    return lax.dot_general(a, b, (((1,), (1,)), ((), ())),
                           preferred_element_type=jnp.float32)


def _xn_body(x_ref, o_ref):
    x = x_ref[...]
    nrm = jnp.maximum(jnp.sqrt(jnp.sum(x * x, axis=1, keepdims=True)), 1e-8)
    o_ref[...] = x / nrm


def _sim_body(xa_ref, xb_ref, o_ref):
    o_ref[...] = lax.bitcast_convert_type(
        jnp.abs(_dot_t(xa_ref[...], xb_ref[...])), jnp.int32)


def _pack_body(s_ref, thr_ref, a_ref, cnt_ref):
    t = thr_ref[0, 0]
    m = s_ref[...] >= t
    a_ref[...] = m.astype(jnp.int8)
    cnt_ref[...] = jnp.maximum(
        jnp.sum(m.astype(jnp.float32), axis=1, keepdims=True), 1.0)


def _inproj_body(x_ref, w_ref, o_ref):
    o_ref[...] = jnp.maximum(_dot_t(x_ref[...], w_ref[...]), 0.0)


def _stats_body(h_ref, m_ref, r_ref):
    h = h_ref[...]
    m = jnp.mean(h, axis=0, keepdims=True)
    v = jnp.mean((h - m) * (h - m), axis=0, keepdims=True)
    m_ref[...] = m
    r_ref[...] = 1.0 / jnp.sqrt(v + 1e-5)


def _bn_body(h_ref, m_ref, r_ref, g_ref, b_ref, o_ref):
    o_ref[...] = g_ref[...] * (h_ref[...] - m_ref[...]) * r_ref[...] + b_ref[...]


def _make_conv_big(mode):
    def body(*refs):
        if mode == "res_relu":
            (a_ref, tj_ref, ti_ref, cnt_ref, wl_ref, bl_ref, wr_ref,
             res_ref, o_ref) = refs
        else:
            a_ref, tj_ref, ti_ref, cnt_ref, wl_ref, bl_ref, wr_ref, o_ref = refs
        j = pl.program_id(1)
        part = jnp.dot(a_ref[...].astype(jnp.float32), tj_ref[...],
                       preferred_element_type=jnp.float32)

        @pl.when(j == 0)
        def _():
            o_ref[...] = part

        @pl.when(j > 0)
        def _():
            o_ref[...] = o_ref[...] + part

        @pl.when(j == NBLK - 1)
        def _():
            mean = o_ref[...] / cnt_ref[...]
            y = (_dot_t(mean, wl_ref[...]) + bl_ref[...]
                 + _dot_t(ti_ref[...], wr_ref[...]))
            if mode == "res_relu":
                y = jnp.maximum(y + res_ref[...], 0.0)
            else:
                y = jnp.maximum(y, 0.0)
            o_ref[...] = y

    return body


_conv_relu_body = _make_conv_big("relu")
_conv_res_body = _make_conv_big("res_relu")


def _head_body(t2_ref, h2_ref, wz_ref, ww_ref, wsc_ref, z_ref, w_ref, r_ref):
    z_ref[...] = _dot_t(t2_ref[...], wz_ref[...])
    w_ref[...] = _dot_t(t2_ref[...], ww_ref[...])
    r_ref[...] = _dot_t(h2_ref[...], wsc_ref[...])


def _cs1_body(a_ref, zj_ref, cnt_ref, bl_ref, wi_ref, o_ref):
    j = pl.program_id(1)
    part = jnp.dot(a_ref[...].astype(jnp.float32), zj_ref[...],
                   preferred_element_type=jnp.float32)

    @pl.when(j == 0)
    def _():
        o_ref[...] = part

    @pl.when(j > 0)
    def _():
        o_ref[...] = o_ref[...] + part

    @pl.when(j == NBLK - 1)
    def _():
        o_ref[...] = jnp.maximum(
            o_ref[...] / cnt_ref[...] + bl_ref[...] + wi_ref[...], 0.0)


def _cs2_body(a_ref, yj_ref, cnt_ref, yi_ref, rsc_ref, wl2_ref, bl2_ref,
              wr2_ref, scb_ref, o_ref):
    j = pl.program_id(1)
    part = jnp.dot(a_ref[...].astype(jnp.float32), yj_ref[...],
                   preferred_element_type=jnp.float32)

    @pl.when(j == 0)
    def _():
        o_ref[...] = part

    @pl.when(j > 0)
    def _():
        o_ref[...] = o_ref[...] + part

    @pl.when(j == NBLK - 1)
    def _():
        y = (o_ref[...] / cnt_ref[...] * wl2_ref[...] + bl2_ref[...]
             + yi_ref[...] * wr2_ref[...] + rsc_ref[...] + scb_ref[...])
        o_ref[...] = 1.0 / (1.0 + jnp.exp(-y))


def _bs_rows(shape):
    return pl.BlockSpec(shape, lambda i: (i, 0))


def _bs_full(shape):
    nd = len(shape)
    return pl.BlockSpec(shape, lambda i: (0,) * nd)


def _conv_specs(feat):
    return dict(
        a=pl.BlockSpec((RB, RB), lambda i, j: (i, j)),
        opj=pl.BlockSpec((RB, feat), lambda i, j: (j, 0)),
        opi=pl.BlockSpec((RB, feat), lambda i, j: (i, 0)),
        cnt=pl.BlockSpec((RB, 1), lambda i, j: (i, 0)),
        row1=pl.BlockSpec((1, feat), lambda i, j: (0, 0)),
        w=pl.BlockSpec((feat, feat), lambda i, j: (0, 0)),
        out=pl.BlockSpec((RB, feat), lambda i, j: (i, 0)),
    )


def kernel(x, W_in, bn1_gamma, bn1_beta, h_s1_Wl, h_s1_bl, h_s1_Wr,
           h_s2_Wl, h_s2_bl, h_s2_Wr, bn2_gamma, bn2_beta,
           o_s1_Wl, o_s1_bl, o_s1_Wr, o_s2_Wl, o_s2_bl, o_s2_Wr,
           o_sc_W, o_sc_b):
    f32 = jnp.float32

    xn = pl.pallas_call(
        _xn_body, grid=(NBLK,),
        in_specs=[_bs_rows((RB, D))], out_specs=_bs_rows((RB, D)),
        out_shape=jax.ShapeDtypeStruct((N, D), f32),
    )(x)

    sim = pl.pallas_call(
        _sim_body, grid=(NBLK, NBLK),
        in_specs=[
            pl.BlockSpec((RB, D), lambda i, j: (i, 0)),
            pl.BlockSpec((RB, D), lambda i, j: (j, 0)),
        ],
        out_specs=pl.BlockSpec((RB, RB), lambda i, j: (i, j)),
        out_shape=jax.ShapeDtypeStruct((N, N), jnp.int32),
    )(xn, xn)

    simflat = sim.reshape(NTOT)
    hist_p1, hist_p2 = _hist_fns()
    zeros16 = jnp.zeros((16,), jnp.int32)
    h1 = hist_p1(simflat, zeros16)
    nb1, _ = _pick_p1(h1, zeros16)
    h2 = hist_p2(simflat, nb1)
    _, thr = _pick_p2(h2, nb1)

    adj8, cnt = pl.pallas_call(
        _pack_body, grid=(NBLK,),
        in_specs=[_bs_rows((RB, N)), _bs_full((1, 128))],
        out_specs=[_bs_rows((RB, N)), _bs_rows((RB, 1))],
        out_shape=[
            jax.ShapeDtypeStruct((N, N), jnp.int8),
            jax.ShapeDtypeStruct((N, 1), f32),
        ],
    )(sim, thr)

    h = pl.pallas_call(
        _inproj_body, grid=(NBLK,),
        in_specs=[_bs_rows((RB, D)), _bs_full((H, D))],
        out_specs=_bs_rows((RB, H)),
        out_shape=jax.ShapeDtypeStruct((N, H), f32),
    )(x, W_in)

    def stats(arr):
        return pl.pallas_call(
            _stats_body, grid=(),
            in_specs=[pl.BlockSpec((N, H), lambda: (0, 0))],
            out_specs=[pl.BlockSpec((1, H), lambda: (0, 0))] * 2,
            out_shape=[jax.ShapeDtypeStruct((1, H), f32)] * 2,
        )(arr)

    def bn_apply(arr, m, r, g, b):
        return pl.pallas_call(
            _bn_body, grid=(NBLK,),
            in_specs=[_bs_rows((RB, H))] + [_bs_full((1, H))] * 4,
            out_specs=_bs_rows((RB, H)),
            out_shape=jax.ShapeDtypeStruct((N, H), f32),
        )(arr, m, r, g, b)

    m1, r1 = stats(h)
    t1 = bn_apply(h, m1, r1, bn1_gamma.reshape(1, H), bn1_beta.reshape(1, H))

    sp = _conv_specs(H)
    y1 = pl.pallas_call(
        _conv_relu_body, grid=(NBLK, NBLK),
        in_specs=[sp["a"], sp["opj"], sp["opi"], sp["cnt"], sp["w"],
                  sp["row1"], sp["w"]],
        out_specs=sp["out"],
        out_shape=jax.ShapeDtypeStruct((N, H), f32),
    )(adj8, t1, t1, cnt, h_s1_Wl, h_s1_bl.reshape(1, H), h_s1_Wr)

    h2v = pl.pallas_call(
        _conv_res_body, grid=(NBLK, NBLK),
        in_specs=[sp["a"], sp["opj"], sp["opi"], sp["cnt"], sp["w"],
                  sp["row1"], sp["w"], sp["opi"]],
        out_specs=sp["out"],
        out_shape=jax.ShapeDtypeStruct((N, H), f32),
    )(adj8, y1, y1, cnt, h_s2_Wl, h_s2_bl.reshape(1, H), h_s2_Wr, h)

    m2, r2 = stats(h2v)
    t2 = bn_apply(h2v, m2, r2, bn2_gamma.reshape(1, H), bn2_beta.reshape(1, H))

    wz = jnp.broadcast_to(o_s1_Wl, (128, H))
    ww = jnp.broadcast_to(o_s1_Wr, (128, H))
    wsc = jnp.broadcast_to(o_sc_W, (128, H))
    zf, wf, rf = pl.pallas_call(
        _head_body, grid=(NBLK,),
        in_specs=[_bs_rows((RB, H)), _bs_rows((RB, H))] + [_bs_full((128, H))] * 3,
        out_specs=[_bs_rows((RB, 128))] * 3,
        out_shape=[jax.ShapeDtypeStruct((N, 128), f32)] * 3,
    )(t2, h2v, wz, ww, wsc)

    sp1 = _conv_specs(128)
    bl1b = jnp.broadcast_to(o_s1_bl.reshape(1, 1), (1, 128))
    y3 = pl.pallas_call(
        _cs1_body, grid=(NBLK, NBLK),
        in_specs=[sp1["a"], sp1["opj"], sp1["cnt"], sp1["row1"], sp1["opi"]],
        out_specs=sp1["out"],
        out_shape=jax.ShapeDtypeStruct((N, 128), f32),
    )(adj8, zf, cnt, bl1b, wf)

    wl2b = jnp.broadcast_to(o_s2_Wl.reshape(1, 1), (1, 128))
    bl2b = jnp.broadcast_to(o_s2_bl.reshape(1, 1), (1, 128))
    wr2b = jnp.broadcast_to(o_s2_Wr.reshape(1, 1), (1, 128))
    scbb = jnp.broadcast_to(o_sc_b.reshape(1, 1), (1, 128))
    outf = pl.pallas_call(
        _cs2_body, grid=(NBLK, NBLK),
        in_specs=[sp1["a"], sp1["opj"], sp1["cnt"], sp1["opi"], sp1["opi"],
                  sp1["row1"], sp1["row1"], sp1["row1"], sp1["row1"]],
        out_specs=sp1["out"],
        out_shape=jax.ShapeDtypeStruct((N, 128), f32),
    )(adj8, y3, cnt, y3, rf, wl2b, bl2b, wr2b, scbb)

    return outf[:, 0]


# speedup vs baseline: 69.4033x; 1.6013x over previous
import functools

import jax
import jax.numpy as jnp
from jax import lax
from jax.experimental import pallas as pl
from jax.experimental.pallas import tpu as pltpu
from jax.experimental.pallas import tpu_sc as plsc

N = 4096
D = 256
H = 256
NTOT = N * N
KIDX = 15938354
BRANK = NTOT - KIDX
BRANK_F = float(BRANK)

NB = 32768
NBA = 32896
NROWS = NBA // 128
SHIFT1 = 15

NC = 2
NS = 16
NW = NC * NS
CHUNK = NTOT // NW
SLICE = 32768
NSLICE = CHUNK // SLICE
UNROLL = 8

RB = 512
NBLK = N // RB


def _make_hist(shift):
    mesh = plsc.VectorSubcoreMesh(
        core_axis_name="c", subcore_axis_name="s", num_cores=NC, num_subcores=NS
    )

    @functools.partial(
        pl.kernel,
        out_type=jax.ShapeDtypeStruct((NW, NBA), jnp.int32),
        mesh=mesh,
        compiler_params=pltpu.CompilerParams(needs_layout_passes=False),
        scratch_types=[
            pltpu.VMEM((NBA,), jnp.int32),
            pltpu.VMEM((SLICE,), jnp.int32),
            pltpu.VMEM((16,), jnp.int32),
        ],
    )
    def hist_k(sim_hbm, base_hbm, out_hbm, hist_v, buf_v, base_v):
        wid = lax.axis_index("s") * NC + lax.axis_index("c")

        @plsc.parallel_loop(0, NBA // 16, unroll=8)
        def _(i):
            hist_v[pl.ds(i * 16, 16)] = jnp.zeros((16,), jnp.int32)

        pltpu.sync_copy(base_hbm, base_v)
        base = base_v[...]
        ones = jnp.ones((16,), jnp.int32)
        chunk0 = wid * CHUNK

        def sbody(s, carry):
            pltpu.sync_copy(sim_hbm.at[pl.ds(chunk0 + s * SLICE, SLICE)], buf_v)

            @plsc.parallel_loop(0, SLICE // 16, unroll=UNROLL)
            def _(i):
                bits = buf_v[pl.ds(i * 16, 16)]
                d = bits - base
                if shift:
                    d = d >> shift
                b = jnp.clip(d + 1, 0, NB + 1)
                plsc.addupdate_scatter(hist_v, [b], ones)

            return carry

        lax.fori_loop(0, NSLICE, sbody, 0)
        pltpu.sync_copy(hist_v, out_hbm.at[wid])

    return hist_k


@functools.lru_cache(maxsize=None)
def _hist_fns():
    return _make_hist(SHIFT1), _make_hist(0)


def _make_pick(shift):
    def pick_body(hist_ref, base_ref, nb_ref, thr_ref):
        tot = hist_ref[0]
        for r in range(1, NW):
            tot = tot + hist_ref[r]
        tot = tot.astype(jnp.float32)

        li = lax.broadcasted_iota(jnp.int32, (128, 128), 0)
        lj = lax.broadcasted_iota(jnp.int32, (128, 128), 1)
        ltri = (li >= lj).astype(jnp.float32)
        sfx_lane = jnp.dot(tot, ltri, preferred_element_type=jnp.float32)

        ri = lax.broadcasted_iota(jnp.int32, (NROWS, NROWS), 0)
        rj = lax.broadcasted_iota(jnp.int32, (NROWS, NROWS), 1)
        sutri = (rj > ri).astype(jnp.float32)
        rows_after = jnp.sum(
            jnp.dot(sutri, tot, preferred_element_type=jnp.float32),
            axis=1, keepdims=True,
        )
        sfx = sfx_lane + rows_after

        jr = lax.broadcasted_iota(jnp.int32, (NROWS, 128), 0)
        jc = lax.broadcasted_iota(jnp.int32, (NROWS, 128), 1)
        jf = (jr * 128 + jc).astype(jnp.float32)
        best = jnp.max(jnp.where(sfx >= BRANK_F, jf, -1.0))
        jstar = best.astype(jnp.int32)

        base0 = base_ref[0]
        newbase = base0 + ((jstar - 1) << shift) if shift else base0 + (jstar - 1)
        nb_ref[...] = jnp.full((16,), newbase, jnp.int32)
        thr_ref[...] = jnp.full((1, 128), newbase, jnp.int32)

    def pick(hists, base16):
        return pl.pallas_call(
            pick_body,
            grid=(),
            in_specs=[
                pl.BlockSpec((NW, NROWS, 128), lambda: (0, 0, 0)),
                pl.BlockSpec((16,), lambda: (0,)),
            ],
            out_specs=[
                pl.BlockSpec((16,), lambda: (0,)),
                pl.BlockSpec((1, 128), lambda: (0, 0)),
            ],
            out_shape=[
                jax.ShapeDtypeStruct((16,), jnp.int32),
                jax.ShapeDtypeStruct((1, 128), jnp.int32),
            ],
        )(hists.reshape(NW, NROWS, 128), base16)

    return pick


_pick_p1 = _make_pick(SHIFT1)
_pick_p2 = _make_pick(0)


def _dot_t(a, b):
    return lax.dot_general(a, b, (((1,), (1,)), ((), ())),
                           preferred_element_type=jnp.float32)


def _xn_body(x_ref, o_ref):
    x = x_ref[...]
    nrm = jnp.maximum(jnp.sqrt(jnp.sum(x * x, axis=1, keepdims=True)), 1e-8)
    o_ref[...] = x / nrm


def _sim_body(xa_ref, xb_ref, o_ref):
    o_ref[...] = lax.bitcast_convert_type(
        jnp.abs(_dot_t(xa_ref[...], xb_ref[...])), jnp.int32)


def _pack_body(s_ref, thr_ref, a_ref, cnt_ref):
    t = thr_ref[0, 0]
    m = s_ref[...] >= t
    a_ref[...] = m.astype(jnp.int8)
    cnt_ref[...] = jnp.maximum(
        jnp.sum(m.astype(jnp.float32), axis=1, keepdims=True), 1.0)


def _inproj_body(x_ref, w_ref, o_ref):
    o_ref[...] = jnp.maximum(_dot_t(x_ref[...], w_ref[...]), 0.0)


def _stats_body(h_ref, m_ref, r_ref):
    h = h_ref[...]
    m = jnp.mean(h, axis=0, keepdims=True)
    v = jnp.mean((h - m) * (h - m), axis=0, keepdims=True)
    m_ref[...] = m
    r_ref[...] = 1.0 / jnp.sqrt(v + 1e-5)


def _bn_body(h_ref, m_ref, r_ref, g_ref, b_ref, o_ref):
    o_ref[...] = g_ref[...] * (h_ref[...] - m_ref[...]) * r_ref[...] + b_ref[...]


def _make_conv_big(mode):
    def body(*refs):
        if mode == "res_relu":
            (a_ref, tj_ref, ti_ref, cnt_ref, wl_ref, bl_ref, wr_ref,
             res_ref, o_ref) = refs
        else:
            a_ref, tj_ref, ti_ref, cnt_ref, wl_ref, bl_ref, wr_ref, o_ref = refs
        j = pl.program_id(1)
        part = jnp.dot(a_ref[...].astype(jnp.float32), tj_ref[...],
                       preferred_element_type=jnp.float32)

        @pl.when(j == 0)
        def _():
            o_ref[...] = part

        @pl.when(j > 0)
        def _():
            o_ref[...] = o_ref[...] + part

        @pl.when(j == NBLK - 1)
        def _():
            mean = o_ref[...] / cnt_ref[...]
            y = (_dot_t(mean, wl_ref[...]) + bl_ref[...]
                 + _dot_t(ti_ref[...], wr_ref[...]))
            if mode == "res_relu":
                y = jnp.maximum(y + res_ref[...], 0.0)
            else:
                y = jnp.maximum(y, 0.0)
            o_ref[...] = y

    return body


_conv_relu_body = _make_conv_big("relu")
_conv_res_body = _make_conv_big("res_relu")


def _head_body(t2_ref, h2_ref, wz_ref, ww_ref, wsc_ref, z_ref, w_ref, r_ref):
    z_ref[...] = _dot_t(t2_ref[...], wz_ref[...])
    w_ref[...] = _dot_t(t2_ref[...], ww_ref[...])
    r_ref[...] = _dot_t(h2_ref[...], wsc_ref[...])


def _cs1_body(a_ref, zj_ref, cnt_ref, bl_ref, wi_ref, o_ref):
    j = pl.program_id(1)
    part = jnp.dot(a_ref[...].astype(jnp.float32), zj_ref[...],
                   preferred_element_type=jnp.float32)

    @pl.when(j == 0)
    def _():
        o_ref[...] = part

    @pl.when(j > 0)
    def _():
        o_ref[...] = o_ref[...] + part

    @pl.when(j == NBLK - 1)
    def _():
        o_ref[...] = jnp.maximum(
            o_ref[...] / cnt_ref[...] + bl_ref[...] + wi_ref[...], 0.0)


def _cs2_body(a_ref, yj_ref, cnt_ref, yi_ref, rsc_ref, wl2_ref, bl2_ref,
              wr2_ref, scb_ref, o_ref):
    j = pl.program_id(1)
    part = jnp.dot(a_ref[...].astype(jnp.float32), yj_ref[...],
                   preferred_element_type=jnp.float32)

    @pl.when(j == 0)
    def _():
        o_ref[...] = part

    @pl.when(j > 0)
    def _():
        o_ref[...] = o_ref[...] + part

    @pl.when(j == NBLK - 1)
    def _():
        y = (o_ref[...] / cnt_ref[...] * wl2_ref[...] + bl2_ref[...]
             + yi_ref[...] * wr2_ref[...] + rsc_ref[...] + scb_ref[...])
        o_ref[...] = 1.0 / (1.0 + jnp.exp(-y))


def _bs_rows(shape):
    return pl.BlockSpec(shape, lambda i: (i, 0))


def _bs_full(shape):
    nd = len(shape)
    return pl.BlockSpec(shape, lambda i: (0,) * nd)


def _conv_specs(feat):
    return dict(
        a=pl.BlockSpec((RB, RB), lambda i, j: (i, j)),
        opj=pl.BlockSpec((RB, feat), lambda i, j: (j, 0)),
        opi=pl.BlockSpec((RB, feat), lambda i, j: (i, 0)),
        cnt=pl.BlockSpec((RB, 1), lambda i, j: (i, 0)),
        row1=pl.BlockSpec((1, feat), lambda i, j: (0, 0)),
        w=pl.BlockSpec((feat, feat), lambda i, j: (0, 0)),
        out=pl.BlockSpec((RB, feat), lambda i, j: (i, 0)),
    )


def kernel(x, W_in, bn1_gamma, bn1_beta, h_s1_Wl, h_s1_bl, h_s1_Wr,
           h_s2_Wl, h_s2_bl, h_s2_Wr, bn2_gamma, bn2_beta,
           o_s1_Wl, o_s1_bl, o_s1_Wr, o_s2_Wl, o_s2_bl, o_s2_Wr,
           o_sc_W, o_sc_b):
    f32 = jnp.float32

    xn = pl.pallas_call(
        _xn_body, grid=(NBLK,),
        in_specs=[_bs_rows((RB, D))], out_specs=_bs_rows((RB, D)),
        out_shape=jax.ShapeDtypeStruct((N, D), f32),
    )(x)

    sim = pl.pallas_call(
        _sim_body, grid=(NBLK, NBLK),
        in_specs=[
            pl.BlockSpec((RB, D), lambda i, j: (i, 0)),
            pl.BlockSpec((RB, D), lambda i, j: (j, 0)),
        ],
        out_specs=pl.BlockSpec((RB, RB), lambda i, j: (i, j)),
        out_shape=jax.ShapeDtypeStruct((N, N), jnp.int32),
    )(xn, xn)

    simflat = sim.reshape(NTOT)
    hist_p1, hist_p2 = _hist_fns()
    zeros16 = jnp.zeros((16,), jnp.int32)
    h1 = hist_p1(simflat, zeros16)
    nb1, _ = _pick_p1(h1, zeros16)
    h2 = hist_p2(simflat, nb1)
    _, thr = _pick_p2(h2, nb1)

    adj8, cnt = pl.pallas_call(
        _pack_body, grid=(NBLK,),
        in_specs=[_bs_rows((RB, N)), _bs_full((1, 128))],
        out_specs=[_bs_rows((RB, N)), _bs_rows((RB, 1))],
        out_shape=[
            jax.ShapeDtypeStruct((N, N), jnp.int8),
            jax.ShapeDtypeStruct((N, 1), f32),
        ],
    )(sim, thr)

    h = pl.pallas_call(
        _inproj_body, grid=(NBLK,),
        in_specs=[_bs_rows((RB, D)), _bs_full((H, D))],
        out_specs=_bs_rows((RB, H)),
        out_shape=jax.ShapeDtypeStruct((N, H), f32),
    )(x, W_in)

    def stats(arr):
        return pl.pallas_call(
            _stats_body, grid=(),
            in_specs=[pl.BlockSpec((N, H), lambda: (0, 0))],
            out_specs=[pl.BlockSpec((1, H), lambda: (0, 0))] * 2,
            out_shape=[jax.ShapeDtypeStruct((1, H), f32)] * 2,
        )(arr)

    def bn_apply(arr, m, r, g, b):
        return pl.pallas_call(
            _bn_body, grid=(NBLK,),
            in_specs=[_bs_rows((RB, H))] + [_bs_full((1, H))] * 4,
            out_specs=_bs_rows((RB, H)),
            out_shape=jax.ShapeDtypeStruct((N, H), f32),
        )(arr, m, r, g, b)

    m1, r1 = stats(h)
    t1 = bn_apply(h, m1, r1, bn1_gamma.reshape(1, H), bn1_beta.reshape(1, H))

    sp = _conv_specs(H)
    y1 = pl.pallas_call(
        _conv_relu_body, grid=(NBLK, NBLK),
        in_specs=[sp["a"], sp["opj"], sp["opi"], sp["cnt"], sp["w"],
                  sp["row1"], sp["w"]],
        out_specs=sp["out"],
        out_shape=jax.ShapeDtypeStruct((N, H), f32),
    )(adj8, t1, t1, cnt, h_s1_Wl, h_s1_bl.reshape(1, H), h_s1_Wr)

    h2v = pl.pallas_call(
        _conv_res_body, grid=(NBLK, NBLK),
        in_specs=[sp["a"], sp["opj"], sp["opi"], sp["cnt"], sp["w"],
                  sp["row1"], sp["w"], sp["opi"]],
        out_specs=sp["out"],
        out_shape=jax.ShapeDtypeStruct((N, H), f32),
    )(adj8, y1, y1, cnt, h_s2_Wl, h_s2_bl.reshape(1, H), h_s2_Wr, h)

    m2, r2 = stats(h2v)
    t2 = bn_apply(h2v, m2, r2, bn2_gamma.reshape(1, H), bn2_beta.reshape(1, H))

    wz = jnp.broadcast_to(o_s1_Wl, (128, H))
    ww = jnp.broadcast_to(o_s1_Wr, (128, H))
    wsc = jnp.broadcast_to(o_sc_W, (128, H))
    zf, wf, rf = pl.pallas_call(
        _head_body, grid=(NBLK,),
        in_specs=[_bs_rows((RB, H)), _bs_rows((RB, H))] + [_bs_full((128, H))] * 3,
        out_specs=[_bs_rows((RB, 128))] * 3,
        out_shape=[jax.ShapeDtypeStruct((N, 128), f32)] * 3,
    )(t2, h2v, wz, ww, wsc)

    sp1 = _conv_specs(128)
    bl1b = jnp.broadcast_to(o_s1_bl.reshape(1, 1), (1, 128))
    y3 = pl.pallas_call(
        _cs1_body, grid=(NBLK, NBLK),
        in_specs=[sp1["a"], sp1["opj"], sp1["cnt"], sp1["row1"], sp1["opi"]],
        out_specs=sp1["out"],
        out_shape=jax.ShapeDtypeStruct((N, 128), f32),
    )(adj8, zf, cnt, bl1b, wf)

    wl2b = jnp.broadcast_to(o_s2_Wl.reshape(1, 1), (1, 128))
    bl2b = jnp.broadcast_to(o_s2_bl.reshape(1, 1), (1, 128))
    wr2b = jnp.broadcast_to(o_s2_Wr.reshape(1, 1), (1, 128))
    scbb = jnp.broadcast_to(o_sc_b.reshape(1, 1), (1, 128))
    outf = pl.pallas_call(
        _cs2_body, grid=(NBLK, NBLK),
        in_specs=[sp1["a"], sp1["opj"], sp1["cnt"], sp1["opi"], sp1["opi"],
                  sp1["row1"], sp1["row1"], sp1["row1"], sp1["row1"]],
        out_specs=sp1["out"],
        out_shape=jax.ShapeDtypeStruct((N, 128), f32),
    )(adj8, y3, cnt, y3, rf, wl2b, bl2b, wr2b, scbb)

    return outf[:, 0]


# speedup vs baseline: 89.1735x; 1.2849x over previous
import functools

import jax
import jax.numpy as jnp
from jax import lax
from jax.experimental import pallas as pl
from jax.experimental.pallas import tpu as pltpu
from jax.experimental.pallas import tpu_sc as plsc

N = 4096
D = 256
H = 256
NTOT = N * N
KIDX = 15938354
BRANK = NTOT - KIDX
BRANK_F = float(BRANK)

NB = 32768
NBA = 32896
NROWS = NBA // 128
SHIFT1 = 15

NC = 2
NS = 16
NW = NC * NS
SLICE = 32768
UNROLL = 8

NDIAG = 8
NUPPER = 28
NBLK_SC = NDIAG + NUPPER
BLKE = 512 * 512
NSC = NBLK_SC * BLKE
CHUNK = NSC // NW
NSLICE = CHUNK // SLICE
DIAG_SLICES = NDIAG * BLKE // SLICE

RB = 512
NBLK = N // RB


def _make_hist(shift):
    mesh = plsc.VectorSubcoreMesh(
        core_axis_name="c", subcore_axis_name="s", num_cores=NC, num_subcores=NS
    )

    @functools.partial(
        pl.kernel,
        out_type=jax.ShapeDtypeStruct((NW, NBA), jnp.int32),
        mesh=mesh,
        compiler_params=pltpu.CompilerParams(needs_layout_passes=False),
        scratch_types=[
            pltpu.VMEM((NBA,), jnp.int32),
            pltpu.VMEM((SLICE,), jnp.int32),
            pltpu.VMEM((16,), jnp.int32),
        ],
    )
    def hist_k(sim_hbm, base_hbm, out_hbm, hist_v, buf_v, base_v):
        wid = lax.axis_index("s") * NC + lax.axis_index("c")

        @plsc.parallel_loop(0, NBA // 16, unroll=8)
        def _(i):
            hist_v[pl.ds(i * 16, 16)] = jnp.zeros((16,), jnp.int32)

        pltpu.sync_copy(base_hbm, base_v)
        base = base_v[...]
        chunk0 = wid * NSLICE

        def sbody(s, carry):
            g = chunk0 + s
            pltpu.sync_copy(sim_hbm.at[pl.ds(g * SLICE, SLICE)], buf_v)
            w = jnp.where(g < DIAG_SLICES, 1, 2).astype(jnp.int32)
            wvec = jnp.zeros((16,), jnp.int32) + w

            @plsc.parallel_loop(0, SLICE // 16, unroll=UNROLL)
            def _(i):
                bits = buf_v[pl.ds(i * 16, 16)]
                d = bits - base
                if shift:
                    d = d >> shift
                b = jnp.clip(d + 1, 0, NB + 1)
                plsc.addupdate_scatter(hist_v, [b], wvec)

            return carry

        lax.fori_loop(0, NSLICE, sbody, 0)
        pltpu.sync_copy(hist_v, out_hbm.at[wid])

    return hist_k


@functools.lru_cache(maxsize=None)
def _hist_fns():
    return _make_hist(SHIFT1), _make_hist(0)


def _make_pick(shift):
    def pick_body(hist_ref, base_ref, nb_ref, thr_ref):
        tot = hist_ref[0]
        for r in range(1, NW):
            tot = tot + hist_ref[r]
        tot = tot.astype(jnp.float32)

        li = lax.broadcasted_iota(jnp.int32, (128, 128), 0)
        lj = lax.broadcasted_iota(jnp.int32, (128, 128), 1)
        ltri = (li >= lj).astype(jnp.float32)
        sfx_lane = jnp.dot(tot, ltri, preferred_element_type=jnp.float32)

        ri = lax.broadcasted_iota(jnp.int32, (NROWS, NROWS), 0)
        rj = lax.broadcasted_iota(jnp.int32, (NROWS, NROWS), 1)
        sutri = (rj > ri).astype(jnp.float32)
        rows_after = jnp.sum(
            jnp.dot(sutri, tot, preferred_element_type=jnp.float32),
            axis=1, keepdims=True,
        )
        sfx = sfx_lane + rows_after

        jr = lax.broadcasted_iota(jnp.int32, (NROWS, 128), 0)
        jc = lax.broadcasted_iota(jnp.int32, (NROWS, 128), 1)
        jf = (jr * 128 + jc).astype(jnp.float32)
        best = jnp.max(jnp.where(sfx >= BRANK_F, jf, -1.0))
        jstar = best.astype(jnp.int32)

        base0 = base_ref[0]
        newbase = base0 + ((jstar - 1) << shift) if shift else base0 + (jstar - 1)
        nb_ref[...] = jnp.full((16,), newbase, jnp.int32)
        thr_ref[...] = jnp.full((1, 128), newbase, jnp.int32)

    def pick(hists, base16):
        return pl.pallas_call(
            pick_body,
            grid=(),
            in_specs=[
                pl.BlockSpec((NW, NROWS, 128), lambda: (0, 0, 0)),
                pl.BlockSpec((16,), lambda: (0,)),
            ],
            out_specs=[
                pl.BlockSpec((16,), lambda: (0,)),
                pl.BlockSpec((1, 128), lambda: (0, 0)),
            ],
            out_shape=[
                jax.ShapeDtypeStruct((16,), jnp.int32),
                jax.ShapeDtypeStruct((1, 128), jnp.int32),
            ],
        )(hists.reshape(NW, NROWS, 128), base16)

    return pick


_pick_p1 = _make_pick(SHIFT1)
_pick_p2 = _make_pick(0)


def _dot_t(a, b):
    return lax.dot_general(a, b, (((1,), (1,)), ((), ())),
                           preferred_element_type=jnp.float32)


def _xn_body(x_ref, o_ref):
    x = x_ref[...]
    nrm = jnp.maximum(jnp.sqrt(jnp.sum(x * x, axis=1, keepdims=True)), 1e-8)
    o_ref[...] = x / nrm


def _sim_body(xa_ref, xb_ref, o_ref, u_ref):
    bits = lax.bitcast_convert_type(
        jnp.abs(_dot_t(xa_ref[...], xb_ref[...])), jnp.int32)
    o_ref[...] = bits
    u_ref[0] = bits


def _upper_slot(i, j):
    off = NDIAG + 7 * i - (i * (i - 1)) // 2 + (j - i - 1)
    return jnp.where(j > i, off, jnp.where(j == i, i, NBLK_SC))


def _pack_body(s_ref, thr_ref, a_ref, cnt_ref):
    t = thr_ref[0, 0]
    m = s_ref[...] >= t
    a_ref[...] = m.astype(jnp.int8)
    cnt_ref[...] = jnp.maximum(
        jnp.sum(m.astype(jnp.float32), axis=1, keepdims=True), 1.0)


def _inproj_body(x_ref, w_ref, o_ref):
    o_ref[...] = jnp.maximum(_dot_t(x_ref[...], w_ref[...]), 0.0)


def _stats_body(h_ref, m_ref, r_ref):
    h = h_ref[...]
    m = jnp.mean(h, axis=0, keepdims=True)
    v = jnp.mean((h - m) * (h - m), axis=0, keepdims=True)
    m_ref[...] = m
    r_ref[...] = 1.0 / jnp.sqrt(v + 1e-5)


def _bn_body(h_ref, m_ref, r_ref, g_ref, b_ref, o_ref):
    o_ref[...] = g_ref[...] * (h_ref[...] - m_ref[...]) * r_ref[...] + b_ref[...]


def _make_conv_big(mode):
    def body(*refs):
        if mode == "res_relu":
            (a_ref, tj_ref, ti_ref, cnt_ref, wl_ref, bl_ref, wr_ref,
             res_ref, o_ref) = refs
        else:
            a_ref, tj_ref, ti_ref, cnt_ref, wl_ref, bl_ref, wr_ref, o_ref = refs
        j = pl.program_id(1)
        part = jnp.dot(a_ref[...].astype(jnp.float32), tj_ref[...],
                       preferred_element_type=jnp.float32)

        @pl.when(j == 0)
        def _():
            o_ref[...] = part

        @pl.when(j > 0)
        def _():
            o_ref[...] = o_ref[...] + part

        @pl.when(j == NBLK - 1)
        def _():
            mean = o_ref[...] / cnt_ref[...]
            y = (_dot_t(mean, wl_ref[...]) + bl_ref[...]
                 + _dot_t(ti_ref[...], wr_ref[...]))
            if mode == "res_relu":
                y = jnp.maximum(y + res_ref[...], 0.0)
            else:
                y = jnp.maximum(y, 0.0)
            o_ref[...] = y

    return body


_conv_relu_body = _make_conv_big("relu")
_conv_res_body = _make_conv_big("res_relu")


def _head_body(t2_ref, h2_ref, wz_ref, ww_ref, wsc_ref, z_ref, w_ref, r_ref):
    z_ref[...] = _dot_t(t2_ref[...], wz_ref[...])
    w_ref[...] = _dot_t(t2_ref[...], ww_ref[...])
    r_ref[...] = _dot_t(h2_ref[...], wsc_ref[...])


def _cs1_body(a_ref, zj_ref, cnt_ref, bl_ref, wi_ref, o_ref):
    j = pl.program_id(1)
    part = jnp.dot(a_ref[...].astype(jnp.float32), zj_ref[...],
                   preferred_element_type=jnp.float32)

    @pl.when(j == 0)
    def _():
        o_ref[...] = part

    @pl.when(j > 0)
    def _():
        o_ref[...] = o_ref[...] + part

    @pl.when(j == NBLK - 1)
    def _():
        o_ref[...] = jnp.maximum(
            o_ref[...] / cnt_ref[...] + bl_ref[...] + wi_ref[...], 0.0)


def _cs2_body(a_ref, yj_ref, cnt_ref, yi_ref, rsc_ref, wl2_ref, bl2_ref,
              wr2_ref, scb_ref, o_ref):
    j = pl.program_id(1)
    part = jnp.dot(a_ref[...].astype(jnp.float32), yj_ref[...],
                   preferred_element_type=jnp.float32)

    @pl.when(j == 0)
    def _():
        o_ref[...] = part

    @pl.when(j > 0)
    def _():
        o_ref[...] = o_ref[...] + part

    @pl.when(j == NBLK - 1)
    def _():
        y = (o_ref[...] / cnt_ref[...] * wl2_ref[...] + bl2_ref[...]
             + yi_ref[...] * wr2_ref[...] + rsc_ref[...] + scb_ref[...])
        o_ref[...] = 1.0 / (1.0 + jnp.exp(-y))


def _bs_rows(shape):
    return pl.BlockSpec(shape, lambda i: (i, 0))


def _bs_full(shape):
    nd = len(shape)
    return pl.BlockSpec(shape, lambda i: (0,) * nd)


def _conv_specs(feat):
    return dict(
        a=pl.BlockSpec((RB, RB), lambda i, j: (i, j)),
        opj=pl.BlockSpec((RB, feat), lambda i, j: (j, 0)),
        opi=pl.BlockSpec((RB, feat), lambda i, j: (i, 0)),
        cnt=pl.BlockSpec((RB, 1), lambda i, j: (i, 0)),
        row1=pl.BlockSpec((1, feat), lambda i, j: (0, 0)),
        w=pl.BlockSpec((feat, feat), lambda i, j: (0, 0)),
        out=pl.BlockSpec((RB, feat), lambda i, j: (i, 0)),
    )


def kernel(x, W_in, bn1_gamma, bn1_beta, h_s1_Wl, h_s1_bl, h_s1_Wr,
           h_s2_Wl, h_s2_bl, h_s2_Wr, bn2_gamma, bn2_beta,
           o_s1_Wl, o_s1_bl, o_s1_Wr, o_s2_Wl, o_s2_bl, o_s2_Wr,
           o_sc_W, o_sc_b):
    f32 = jnp.float32

    xn = pl.pallas_call(
        _xn_body, grid=(NBLK,),
        in_specs=[_bs_rows((RB, D))], out_specs=_bs_rows((RB, D)),
        out_shape=jax.ShapeDtypeStruct((N, D), f32),
    )(x)

    sim, upper = pl.pallas_call(
        _sim_body, grid=(NBLK, NBLK),
        in_specs=[
            pl.BlockSpec((RB, D), lambda i, j: (i, 0)),
            pl.BlockSpec((RB, D), lambda i, j: (j, 0)),
        ],
        out_specs=[
            pl.BlockSpec((RB, RB), lambda i, j: (i, j)),
            pl.BlockSpec((1, RB, RB), lambda i, j: (_upper_slot(i, j), 0, 0)),
        ],
        out_shape=[
            jax.ShapeDtypeStruct((N, N), jnp.int32),
            jax.ShapeDtypeStruct((NBLK_SC + 1, RB, RB), jnp.int32),
        ],
    )(xn, xn)

    ubits = upper.reshape((NBLK_SC + 1) * BLKE)
    hist_p1, hist_p2 = _hist_fns()
    zeros16 = jnp.zeros((16,), jnp.int32)
    h1 = hist_p1(ubits, zeros16)
    nb1, _ = _pick_p1(h1, zeros16)
    h2 = hist_p2(ubits, nb1)
    _, thr = _pick_p2(h2, nb1)

    adj8, cnt = pl.pallas_call(
        _pack_body, grid=(NBLK,),
        in_specs=[_bs_rows((RB, N)), _bs_full((1, 128))],
        out_specs=[_bs_rows((RB, N)), _bs_rows((RB, 1))],
        out_shape=[
            jax.ShapeDtypeStruct((N, N), jnp.int8),
            jax.ShapeDtypeStruct((N, 1), f32),
        ],
    )(sim, thr)

    h = pl.pallas_call(
        _inproj_body, grid=(NBLK,),
        in_specs=[_bs_rows((RB, D)), _bs_full((H, D))],
        out_specs=_bs_rows((RB, H)),
        out_shape=jax.ShapeDtypeStruct((N, H), f32),
    )(x, W_in)

    def stats(arr):
        return pl.pallas_call(
            _stats_body, grid=(),
            in_specs=[pl.BlockSpec((N, H), lambda: (0, 0))],
            out_specs=[pl.BlockSpec((1, H), lambda: (0, 0))] * 2,
            out_shape=[jax.ShapeDtypeStruct((1, H), f32)] * 2,
        )(arr)

    def bn_apply(arr, m, r, g, b):
        return pl.pallas_call(
            _bn_body, grid=(NBLK,),
            in_specs=[_bs_rows((RB, H))] + [_bs_full((1, H))] * 4,
            out_specs=_bs_rows((RB, H)),
            out_shape=jax.ShapeDtypeStruct((N, H), f32),
        )(arr, m, r, g, b)

    m1, r1 = stats(h)
    t1 = bn_apply(h, m1, r1, bn1_gamma.reshape(1, H), bn1_beta.reshape(1, H))

    sp = _conv_specs(H)
    y1 = pl.pallas_call(
        _conv_relu_body, grid=(NBLK, NBLK),
        in_specs=[sp["a"], sp["opj"], sp["opi"], sp["cnt"], sp["w"],
                  sp["row1"], sp["w"]],
        out_specs=sp["out"],
        out_shape=jax.ShapeDtypeStruct((N, H), f32),
    )(adj8, t1, t1, cnt, h_s1_Wl, h_s1_bl.reshape(1, H), h_s1_Wr)

    h2v = pl.pallas_call(
        _conv_res_body, grid=(NBLK, NBLK),
        in_specs=[sp["a"], sp["opj"], sp["opi"], sp["cnt"], sp["w"],
                  sp["row1"], sp["w"], sp["opi"]],
        out_specs=sp["out"],
        out_shape=jax.ShapeDtypeStruct((N, H), f32),
    )(adj8, y1, y1, cnt, h_s2_Wl, h_s2_bl.reshape(1, H), h_s2_Wr, h)

    m2, r2 = stats(h2v)
    t2 = bn_apply(h2v, m2, r2, bn2_gamma.reshape(1, H), bn2_beta.reshape(1, H))

    wz = jnp.broadcast_to(o_s1_Wl, (128, H))
    ww = jnp.broadcast_to(o_s1_Wr, (128, H))
    wsc = jnp.broadcast_to(o_sc_W, (128, H))
    zf, wf, rf = pl.pallas_call(
        _head_body, grid=(NBLK,),
        in_specs=[_bs_rows((RB, H)), _bs_rows((RB, H))] + [_bs_full((128, H))] * 3,
        out_specs=[_bs_rows((RB, 128))] * 3,
        out_shape=[jax.ShapeDtypeStruct((N, 128), f32)] * 3,
    )(t2, h2v, wz, ww, wsc)

    sp1 = _conv_specs(128)
    bl1b = jnp.broadcast_to(o_s1_bl.reshape(1, 1), (1, 128))
    y3 = pl.pallas_call(
        _cs1_body, grid=(NBLK, NBLK),
        in_specs=[sp1["a"], sp1["opj"], sp1["cnt"], sp1["row1"], sp1["opi"]],
        out_specs=sp1["out"],
        out_shape=jax.ShapeDtypeStruct((N, 128), f32),
    )(adj8, zf, cnt, bl1b, wf)

    wl2b = jnp.broadcast_to(o_s2_Wl.reshape(1, 1), (1, 128))
    bl2b = jnp.broadcast_to(o_s2_bl.reshape(1, 1), (1, 128))
    wr2b = jnp.broadcast_to(o_s2_Wr.reshape(1, 1), (1, 128))
    scbb = jnp.broadcast_to(o_sc_b.reshape(1, 1), (1, 128))
    outf = pl.pallas_call(
        _cs2_body, grid=(NBLK, NBLK),
        in_specs=[sp1["a"], sp1["opj"], sp1["cnt"], sp1["opi"], sp1["opi"],
                  sp1["row1"], sp1["row1"], sp1["row1"], sp1["row1"]],
        out_specs=sp1["out"],
        out_shape=jax.ShapeDtypeStruct((N, 128), f32),
    )(adj8, y3, cnt, y3, rf, wl2b, bl2b, wr2b, scbb)

    return outf[:, 0]
